# baseline (speedup 1.0000x reference)
import functools
import jax
import jax.numpy as jnp
from jax import lax
from jax.experimental import pallas as pl
from jax.experimental.pallas import tpu as pltpu
from jax.experimental.pallas import tpu_sc as plsc

S_CHUNK = 2


def kernel(x, pos_table):
    seq_len, batch, d_model = x.shape
    info = plsc.get_sparse_core_info()
    nc, ns, lanes = info.num_cores, info.num_subcores, info.num_lanes
    nw = nc * ns
    seq_pw = seq_len // nw
    n_chunks = seq_pw // S_CHUNK
    nj = d_model // lanes

    @functools.partial(
        pl.kernel,
        mesh=plsc.VectorSubcoreMesh(core_axis_name="c", subcore_axis_name="s"),
        out_type=jax.ShapeDtypeStruct((seq_len, batch, d_model), jnp.float32),
        scratch_types=[
            pltpu.VMEM((2, S_CHUNK, batch, d_model), jnp.float32),
            pltpu.VMEM((seq_pw, d_model), jnp.float32),
            pltpu.VMEM((2, S_CHUNK, batch, d_model), jnp.float32),
            pltpu.SemaphoreType.DMA,
            pltpu.SemaphoreType.DMA,
            pltpu.SemaphoreType.DMA,
            pltpu.SemaphoreType.DMA,
            pltpu.SemaphoreType.DMA,
        ],
    )
    def k(x_hbm, pos_hbm, out_hbm, xbuf, pbuf, obuf,
          xs0, xs1, psem, os0, os1):
        wid = lax.axis_index("s") * nc + lax.axis_index("c")
        seq_base = wid * seq_pw

        xsems = (xs0, xs1)
        osems = (os0, os1)

        def start_load(g):
            b = g % 2
            s0 = seq_base + g * S_CHUNK
            pltpu.async_copy(x_hbm.at[pl.ds(s0, S_CHUNK)], xbuf.at[b],
                             xsems[b])

        pltpu.async_copy(pos_hbm.at[pl.ds(seq_base, seq_pw)], pbuf, psem)
        start_load(0)
        start_load(1)
        pltpu.make_async_copy(pos_hbm.at[pl.ds(seq_base, seq_pw)], pbuf,
                              psem).wait()

        out_started = [False, False]
        for g in range(n_chunks):
            b = g % 2
            s0 = seq_base + g * S_CHUNK
            pltpu.make_async_copy(x_hbm.at[pl.ds(s0, S_CHUNK)], xbuf.at[b],
                                  xsems[b]).wait()
            if out_started[b]:
                prev0 = seq_base + (g - 2) * S_CHUNK
                pltpu.make_async_copy(obuf.at[b],
                                      out_hbm.at[pl.ds(prev0, S_CHUNK)],
                                      osems[b]).wait()

            def body(j, _):
                for s in range(S_CHUNK):
                    p = pbuf[g * S_CHUNK + s, pl.ds(j * lanes, lanes)]
                    for bb in range(batch):
                        obuf[b, s, bb, pl.ds(j * lanes, lanes)] = (
                            xbuf[b, s, bb, pl.ds(j * lanes, lanes)] + p)
                return 0

            lax.fori_loop(0, nj, body, 0)

            pltpu.async_copy(obuf.at[b], out_hbm.at[pl.ds(s0, S_CHUNK)],
                             osems[b])
            out_started[b] = True
            if g + 2 < n_chunks:
                start_load(g + 2)

        for g in (n_chunks - 2, n_chunks - 1):
            b = g % 2
            s0 = seq_base + g * S_CHUNK
            pltpu.make_async_copy(obuf.at[b], out_hbm.at[pl.ds(s0, S_CHUNK)],
                                  osems[b]).wait()

    return k(x, pos_table)


# speedup vs baseline: 1.9127x; 1.9127x over previous
import functools
import jax
import jax.numpy as jnp
from jax import lax
from jax.experimental import pallas as pl
from jax.experimental.pallas import tpu as pltpu
from jax.experimental.pallas import tpu_sc as plsc

S_CHUNK = 4
NIN = 3
NOUT = 2


def kernel(x, pos_table):
    seq_len, batch, d_model = x.shape
    info = plsc.get_sparse_core_info()
    nc, ns, lanes = info.num_cores, info.num_subcores, info.num_lanes
    nw = nc * ns
    seq_pw = seq_len // nw
    n_chunks = seq_pw // S_CHUNK
    nj = d_model // lanes

    @functools.partial(
        pl.kernel,
        mesh=plsc.VectorSubcoreMesh(core_axis_name="c", subcore_axis_name="s"),
        out_type=jax.ShapeDtypeStruct((seq_len, batch, d_model), jnp.float32),
        scratch_types=[
            pltpu.VMEM((NIN, S_CHUNK, batch, d_model), jnp.float32),
            pltpu.VMEM((NIN, S_CHUNK, d_model), jnp.float32),
            pltpu.VMEM((NOUT, S_CHUNK, batch, d_model), jnp.float32),
            pltpu.SemaphoreType.DMA,
            pltpu.SemaphoreType.DMA,
            pltpu.SemaphoreType.DMA,
            pltpu.SemaphoreType.DMA,
            pltpu.SemaphoreType.DMA,
            pltpu.SemaphoreType.DMA,
            pltpu.SemaphoreType.DMA,
            pltpu.SemaphoreType.DMA,
        ],
    )
    def k(x_hbm, pos_hbm, out_hbm, xbuf, pbuf, obuf,
          xs0, xs1, xs2, ps0, ps1, ps2, os0, os1):
        wid = lax.axis_index("s") * nc + lax.axis_index("c")
        seq_base = wid * seq_pw

        xsems = (xs0, xs1, xs2)
        psems = (ps0, ps1, ps2)
        osems = (os0, os1)

        def start_load(g):
            b = g % NIN
            s0 = seq_base + g * S_CHUNK
            pltpu.async_copy(x_hbm.at[pl.ds(s0, S_CHUNK)], xbuf.at[b],
                             xsems[b])
            pltpu.async_copy(pos_hbm.at[pl.ds(s0, S_CHUNK)], pbuf.at[b],
                             psems[b])

        start_load(0)
        start_load(1)
        start_load(2)

        out_started = [False, False]
        for g in range(n_chunks):
            b = g % NIN
            ob = g % NOUT
            s0 = seq_base + g * S_CHUNK
            pltpu.make_async_copy(x_hbm.at[pl.ds(s0, S_CHUNK)], xbuf.at[b],
                                  xsems[b]).wait()
            pltpu.make_async_copy(pos_hbm.at[pl.ds(s0, S_CHUNK)], pbuf.at[b],
                                  psems[b]).wait()
            if out_started[ob]:
                prev0 = seq_base + (g - NOUT) * S_CHUNK
                pltpu.make_async_copy(obuf.at[ob],
                                      out_hbm.at[pl.ds(prev0, S_CHUNK)],
                                      osems[ob]).wait()

            def body(j, _):
                for s in range(S_CHUNK):
                    p = pbuf[b, s, pl.ds(j * lanes, lanes)]
                    for bb in range(batch):
                        obuf[ob, s, bb, pl.ds(j * lanes, lanes)] = (
                            xbuf[b, s, bb, pl.ds(j * lanes, lanes)] + p)
                return 0

            lax.fori_loop(0, nj, body, 0)

            pltpu.async_copy(obuf.at[ob], out_hbm.at[pl.ds(s0, S_CHUNK)],
                             osems[ob])
            out_started[ob] = True
            if g + NIN < n_chunks:
                start_load(g + NIN)

        for g in (n_chunks - 2, n_chunks - 1):
            ob = g % NOUT
            s0 = seq_base + g * S_CHUNK
            pltpu.make_async_copy(obuf.at[ob], out_hbm.at[pl.ds(s0, S_CHUNK)],
                                  osems[ob]).wait()

    return k(x, pos_table)
